# Initial kernel scaffold; baseline (speedup 1.0000x reference)
import jax
import jax.numpy as jnp
from jax.experimental import pallas as pl
from jax.experimental.pallas import tpu as pltpu

NUM_EXPERTS = 8
TOP_K = 2
N_TOKENS = 2048
D_IN = 768
D_OUT = 768


def _moe_body(x_ref, w_ref, b_ref, gw_ref, gb_ref, gamma_ref, beta_ref,
              out_ref, c_ref, xbf_ref):
    e = pl.program_id(0)

    @pl.when(e == 0)
    def _gating():
        x = x_ref[...]
        logits = jnp.dot(x, gw_ref[...],
                         preferred_element_type=jnp.float32) + gb_ref[...]
        gx = jnp.sqrt(jnp.sum(logits * logits, axis=1, keepdims=True))
        nx = gx / (jnp.mean(gx, axis=0, keepdims=True) + 1e-06)
        logits = gamma_ref[...] * (logits * nx) + beta_ref[...] + logits
        m = jnp.max(logits, axis=1, keepdims=True)
        ex = jnp.exp(logits - m)
        gates = ex / jnp.sum(ex, axis=1, keepdims=True)
        ids = jax.lax.broadcasted_iota(jnp.int32, gates.shape, 1)
        m1 = jnp.max(gates, axis=1, keepdims=True)
        i1 = jnp.min(jnp.where(gates == m1, ids, NUM_EXPERTS),
                     axis=1, keepdims=True)
        sel1 = ids == i1
        g2 = jnp.where(sel1, -jnp.inf, gates)
        m2 = jnp.max(g2, axis=1, keepdims=True)
        i2 = jnp.min(jnp.where(g2 == m2, ids, NUM_EXPERTS),
                     axis=1, keepdims=True)
        sel2 = ids == i2
        c_ref[...] = jnp.where(sel1, m1, 0.0) + jnp.where(sel2, m2, 0.0)
        xbf_ref[...] = x.astype(jnp.bfloat16)

    ce = c_ref[:, pl.ds(e, 1)]
    wbf = w_ref[0].astype(jnp.bfloat16)
    y = jnp.dot(xbf_ref[...], wbf, preferred_element_type=jnp.float32)
    contrib = ce * (y + b_ref[0][None, :])

    @pl.when(e == 0)
    def _init():
        out_ref[...] = contrib

    @pl.when(e > 0)
    def _acc():
        out_ref[...] += contrib


@jax.jit
def kernel(x, W, b, gate_W, gate_b, gamma, beta):
    grid = (NUM_EXPERTS,)
    return pl.pallas_call(
        _moe_body,
        grid=grid,
        in_specs=[
            pl.BlockSpec((N_TOKENS, D_IN), lambda e: (0, 0)),
            pl.BlockSpec((1, D_IN, D_OUT), lambda e: (e, 0, 0)),
            pl.BlockSpec((1, D_OUT), lambda e: (e, 0)),
            pl.BlockSpec((D_IN, NUM_EXPERTS), lambda e: (0, 0)),
            pl.BlockSpec((NUM_EXPERTS,), lambda e: (0,)),
            pl.BlockSpec((1, NUM_EXPERTS), lambda e: (0, 0)),
            pl.BlockSpec((1, NUM_EXPERTS), lambda e: (0, 0)),
        ],
        out_specs=pl.BlockSpec((N_TOKENS, D_OUT), lambda e: (0, 0)),
        out_shape=jax.ShapeDtypeStruct((N_TOKENS, D_OUT), jnp.float32),
        scratch_shapes=[
            pltpu.VMEM((N_TOKENS, NUM_EXPERTS), jnp.float32),
            pltpu.VMEM((N_TOKENS, D_IN), jnp.bfloat16),
        ],
    )(x, W, b, gate_W, gate_b, gamma, beta)


# speedup vs baseline: 1.0027x; 1.0027x over previous
import jax
import jax.numpy as jnp
from jax.experimental import pallas as pl
from jax.experimental.pallas import tpu as pltpu

NUM_EXPERTS = 8
TOP_K = 2
N_TOKENS = 2048
D_IN = 768
D_OUT = 768


def _moe_body(x_ref, w_ref, b_ref, gw_ref, gb_ref, gamma_ref, beta_ref,
              out_ref, c_ref, xbf_ref):
    e = pl.program_id(0)

    @pl.when(e == 0)
    def _gating():
        x = x_ref[...]
        logits = jnp.dot(x, gw_ref[...],
                         preferred_element_type=jnp.float32) + gb_ref[...]
        gx = jnp.sqrt(jnp.sum(logits * logits, axis=1, keepdims=True))
        nx = gx / (jnp.mean(gx, axis=0, keepdims=True) + 1e-06)
        logits = gamma_ref[...] * (logits * nx) + beta_ref[...] + logits
        m = jnp.max(logits, axis=1, keepdims=True)
        ex = jnp.exp(logits - m)
        gates = ex / jnp.sum(ex, axis=1, keepdims=True)
        ids = jax.lax.broadcasted_iota(jnp.int32, gates.shape, 1)
        m1 = jnp.max(gates, axis=1, keepdims=True)
        i1 = jnp.min(jnp.where(gates == m1, ids, NUM_EXPERTS),
                     axis=1, keepdims=True)
        sel1 = ids == i1
        g2 = jnp.where(sel1, -jnp.inf, gates)
        m2 = jnp.max(g2, axis=1, keepdims=True)
        i2 = jnp.min(jnp.where(g2 == m2, ids, NUM_EXPERTS),
                     axis=1, keepdims=True)
        sel2 = ids == i2
        c_ref[...] = jnp.where(sel1, m1, 0.0) + jnp.where(sel2, m2, 0.0)
        xbf_ref[...] = x.astype(jnp.bfloat16)

    cids = jax.lax.broadcasted_iota(jnp.int32, (N_TOKENS, NUM_EXPERTS), 1)
    ce = jnp.sum(jnp.where(cids == e, c_ref[...], 0.0), axis=1, keepdims=True)
    wbf = w_ref[0].astype(jnp.bfloat16)
    y = jnp.dot(xbf_ref[...], wbf, preferred_element_type=jnp.float32)
    contrib = ce * (y + b_ref[0])

    @pl.when(e == 0)
    def _init():
        out_ref[...] = contrib

    @pl.when(e > 0)
    def _acc():
        out_ref[...] += contrib


@jax.jit
def kernel(x, W, b, gate_W, gate_b, gamma, beta):
    grid = (NUM_EXPERTS,)
    return pl.pallas_call(
        _moe_body,
        grid=grid,
        in_specs=[
            pl.BlockSpec((N_TOKENS, D_IN), lambda e: (0, 0)),
            pl.BlockSpec((1, D_IN, D_OUT), lambda e: (e, 0, 0)),
            pl.BlockSpec((1, 1, D_OUT), lambda e: (e, 0, 0)),
            pl.BlockSpec((D_IN, NUM_EXPERTS), lambda e: (0, 0)),
            pl.BlockSpec((NUM_EXPERTS,), lambda e: (0,)),
            pl.BlockSpec((1, NUM_EXPERTS), lambda e: (0, 0)),
            pl.BlockSpec((1, NUM_EXPERTS), lambda e: (0, 0)),
        ],
        out_specs=pl.BlockSpec((N_TOKENS, D_OUT), lambda e: (0, 0)),
        out_shape=jax.ShapeDtypeStruct((N_TOKENS, D_OUT), jnp.float32),
        scratch_shapes=[
            pltpu.VMEM((N_TOKENS, NUM_EXPERTS), jnp.float32),
            pltpu.VMEM((N_TOKENS, D_IN), jnp.bfloat16),
        ],
    )(x, W, b.reshape(NUM_EXPERTS, 1, D_OUT), gate_W, gate_b, gamma, beta)


# speedup vs baseline: 1.0643x; 1.0615x over previous
import jax
import jax.numpy as jnp
from jax.experimental import pallas as pl
from jax.experimental.pallas import tpu as pltpu

NUM_EXPERTS = 8
TOP_K = 2
N_TOKENS = 2048
D_IN = 768
D_OUT = 768
BLK = 256
N_BLKS = N_TOKENS // BLK


def _moe_body(x_ref, w_ref, b_ref, gw_ref, gb_ref, gamma_ref, beta_ref,
              out_ref, c_ref, wbf_ref):
    t = pl.program_id(0)

    @pl.when(t == 0)
    def _gating():
        x = x_ref[...]
        logits = jnp.dot(x, gw_ref[...],
                         preferred_element_type=jnp.float32) + gb_ref[...]
        gx = jnp.sqrt(jnp.sum(logits * logits, axis=1, keepdims=True))
        nx = gx / (jnp.mean(gx, axis=0, keepdims=True) + 1e-06)
        logits = gamma_ref[...] * (logits * nx) + beta_ref[...] + logits
        m = jnp.max(logits, axis=1, keepdims=True)
        ex = jnp.exp(logits - m)
        gates = ex / jnp.sum(ex, axis=1, keepdims=True)
        ids = jax.lax.broadcasted_iota(jnp.int32, gates.shape, 1)
        m1 = jnp.max(gates, axis=1, keepdims=True)
        i1 = jnp.min(jnp.where(gates == m1, ids, NUM_EXPERTS),
                     axis=1, keepdims=True)
        sel1 = ids == i1
        g2 = jnp.where(sel1, -jnp.inf, gates)
        m2 = jnp.max(g2, axis=1, keepdims=True)
        i2 = jnp.min(jnp.where(g2 == m2, ids, NUM_EXPERTS),
                     axis=1, keepdims=True)
        sel2 = ids == i2
        c_ref[...] = jnp.where(sel1, m1, 0.0) + jnp.where(sel2, m2, 0.0)
        wbf_ref[...] = w_ref[...].astype(jnp.bfloat16)

    xf = x_ref[pl.ds(t * BLK, BLK), :]
    c = c_ref[pl.ds(t * BLK, BLK), :]
    xs = jnp.concatenate(
        [(c[:, e:e + 1] * xf).astype(jnp.bfloat16)
         for e in range(NUM_EXPERTS)], axis=1)
    y = jnp.dot(xs, wbf_ref[...], preferred_element_type=jnp.float32)
    out_ref[...] = y + jnp.dot(c, b_ref[...],
                               preferred_element_type=jnp.float32)


@jax.jit
def kernel(x, W, b, gate_W, gate_b, gamma, beta):
    ek = NUM_EXPERTS * D_IN
    return pl.pallas_call(
        _moe_body,
        grid=(N_BLKS,),
        in_specs=[
            pl.BlockSpec((N_TOKENS, D_IN), lambda t: (0, 0)),
            pl.BlockSpec((ek, D_OUT), lambda t: (0, 0)),
            pl.BlockSpec((NUM_EXPERTS, D_OUT), lambda t: (0, 0)),
            pl.BlockSpec((D_IN, NUM_EXPERTS), lambda t: (0, 0)),
            pl.BlockSpec((NUM_EXPERTS,), lambda t: (0,)),
            pl.BlockSpec((1, NUM_EXPERTS), lambda t: (0, 0)),
            pl.BlockSpec((1, NUM_EXPERTS), lambda t: (0, 0)),
        ],
        out_specs=pl.BlockSpec((BLK, D_OUT), lambda t: (t, 0)),
        out_shape=jax.ShapeDtypeStruct((N_TOKENS, D_OUT), jnp.float32),
        scratch_shapes=[
            pltpu.VMEM((N_TOKENS, NUM_EXPERTS), jnp.float32),
            pltpu.VMEM((ek, D_OUT), jnp.bfloat16),
        ],
    )(x, W.reshape(ek, D_OUT), b, gate_W, gate_b, gamma, beta)


# speedup vs baseline: 1.0918x; 1.0258x over previous
import jax
import jax.numpy as jnp
from jax.experimental import pallas as pl
from jax.experimental.pallas import tpu as pltpu

NUM_EXPERTS = 8
TOP_K = 2
N_TOKENS = 2048
D_IN = 768
D_OUT = 768
BLK = 512
N_BLKS = N_TOKENS // BLK


def _moe_body(x_ref, w_ref, b_ref, gw_ref, gb_ref, gamma_ref, beta_ref,
              out_ref, c_ref, wbf_ref):
    t = pl.program_id(0)

    @pl.when(t == 0)
    def _gating():
        x = x_ref[...]
        logits = jnp.dot(x, gw_ref[...],
                         preferred_element_type=jnp.float32) + gb_ref[...]
        gx = jnp.sqrt(jnp.sum(logits * logits, axis=1, keepdims=True))
        nx = gx / (jnp.mean(gx, axis=0, keepdims=True) + 1e-06)
        logits = gamma_ref[...] * (logits * nx) + beta_ref[...] + logits
        m = jnp.max(logits, axis=1, keepdims=True)
        ex = jnp.exp(logits - m)
        gates = ex / jnp.sum(ex, axis=1, keepdims=True)
        ids = jax.lax.broadcasted_iota(jnp.int32, gates.shape, 1)
        m1 = jnp.max(gates, axis=1, keepdims=True)
        i1 = jnp.min(jnp.where(gates == m1, ids, NUM_EXPERTS),
                     axis=1, keepdims=True)
        sel1 = ids == i1
        g2 = jnp.where(sel1, -jnp.inf, gates)
        m2 = jnp.max(g2, axis=1, keepdims=True)
        i2 = jnp.min(jnp.where(g2 == m2, ids, NUM_EXPERTS),
                     axis=1, keepdims=True)
        sel2 = ids == i2
        c_ref[...] = jnp.where(sel1, m1, 0.0) + jnp.where(sel2, m2, 0.0)
        wbf_ref[...] = w_ref[...].astype(jnp.bfloat16)

    xf = x_ref[pl.ds(t * BLK, BLK), :]
    c = c_ref[pl.ds(t * BLK, BLK), :]
    xs = jnp.concatenate(
        [(c[:, e:e + 1] * xf).astype(jnp.bfloat16)
         for e in range(NUM_EXPERTS)], axis=1)
    y = jnp.dot(xs, wbf_ref[...], preferred_element_type=jnp.float32)
    out_ref[...] = y + jnp.dot(c, b_ref[...],
                               preferred_element_type=jnp.float32)


@jax.jit
def kernel(x, W, b, gate_W, gate_b, gamma, beta):
    ek = NUM_EXPERTS * D_IN
    return pl.pallas_call(
        _moe_body,
        grid=(N_BLKS,),
        in_specs=[
            pl.BlockSpec((N_TOKENS, D_IN), lambda t: (0, 0)),
            pl.BlockSpec((ek, D_OUT), lambda t: (0, 0)),
            pl.BlockSpec((NUM_EXPERTS, D_OUT), lambda t: (0, 0)),
            pl.BlockSpec((D_IN, NUM_EXPERTS), lambda t: (0, 0)),
            pl.BlockSpec((NUM_EXPERTS,), lambda t: (0,)),
            pl.BlockSpec((1, NUM_EXPERTS), lambda t: (0, 0)),
            pl.BlockSpec((1, NUM_EXPERTS), lambda t: (0, 0)),
        ],
        out_specs=pl.BlockSpec((BLK, D_OUT), lambda t: (t, 0)),
        out_shape=jax.ShapeDtypeStruct((N_TOKENS, D_OUT), jnp.float32),
        scratch_shapes=[
            pltpu.VMEM((N_TOKENS, NUM_EXPERTS), jnp.float32),
            pltpu.VMEM((ek, D_OUT), jnp.bfloat16),
        ],
    )(x, W.reshape(ek, D_OUT), b, gate_W, gate_b, gamma, beta)


# speedup vs baseline: 1.1052x; 1.0123x over previous
import jax
import jax.numpy as jnp
from jax.experimental import pallas as pl
from jax.experimental.pallas import tpu as pltpu

NUM_EXPERTS = 8
TOP_K = 2
N_TOKENS = 2048
D_IN = 768
D_OUT = 768


def _moe_body(x_ref, w_ref, b_ref, gw_ref, gb_ref, gamma_ref, beta_ref,
              out_ref, c_ref, xbf_ref):
    e = pl.program_id(0)

    @pl.when(e == 0)
    def _gating():
        x = x_ref[...]
        logits = jnp.dot(x, gw_ref[...],
                         preferred_element_type=jnp.float32) + gb_ref[...]
        gx = jnp.sqrt(jnp.sum(logits * logits, axis=1, keepdims=True))
        nx = gx / (jnp.mean(gx, axis=0, keepdims=True) + 1e-06)
        logits = gamma_ref[...] * (logits * nx) + beta_ref[...] + logits
        m = jnp.max(logits, axis=1, keepdims=True)
        ex = jnp.exp(logits - m)
        gates = ex / jnp.sum(ex, axis=1, keepdims=True)
        ids = jax.lax.broadcasted_iota(jnp.int32, gates.shape, 1)
        m1 = jnp.max(gates, axis=1, keepdims=True)
        i1 = jnp.min(jnp.where(gates == m1, ids, NUM_EXPERTS),
                     axis=1, keepdims=True)
        sel1 = ids == i1
        g2 = jnp.where(sel1, -jnp.inf, gates)
        m2 = jnp.max(g2, axis=1, keepdims=True)
        i2 = jnp.min(jnp.where(g2 == m2, ids, NUM_EXPERTS),
                     axis=1, keepdims=True)
        sel2 = ids == i2
        c = jnp.where(sel1, m1, 0.0) + jnp.where(sel2, m2, 0.0)
        c_ref[...] = c
        xbf_ref[...] = x.astype(jnp.bfloat16)
        out_ref[...] = jnp.dot(c, b_ref[...],
                               preferred_element_type=jnp.float32)

    cids = jax.lax.broadcasted_iota(jnp.int32, (N_TOKENS, NUM_EXPERTS), 1)
    ce = jnp.sum(jnp.where(cids == e, c_ref[...], 0.0), axis=1, keepdims=True)
    xs = ce.astype(jnp.bfloat16) * xbf_ref[...]
    wbf = w_ref[0].astype(jnp.bfloat16)
    out_ref[...] += jnp.dot(xs, wbf, preferred_element_type=jnp.float32)


@jax.jit
def kernel(x, W, b, gate_W, gate_b, gamma, beta):
    return pl.pallas_call(
        _moe_body,
        grid=(NUM_EXPERTS,),
        in_specs=[
            pl.BlockSpec((N_TOKENS, D_IN), lambda e: (0, 0)),
            pl.BlockSpec((1, D_IN, D_OUT), lambda e: (e, 0, 0)),
            pl.BlockSpec((NUM_EXPERTS, D_OUT), lambda e: (0, 0)),
            pl.BlockSpec((D_IN, NUM_EXPERTS), lambda e: (0, 0)),
            pl.BlockSpec((NUM_EXPERTS,), lambda e: (0,)),
            pl.BlockSpec((1, NUM_EXPERTS), lambda e: (0, 0)),
            pl.BlockSpec((1, NUM_EXPERTS), lambda e: (0, 0)),
        ],
        out_specs=pl.BlockSpec((N_TOKENS, D_OUT), lambda e: (0, 0)),
        out_shape=jax.ShapeDtypeStruct((N_TOKENS, D_OUT), jnp.float32),
        scratch_shapes=[
            pltpu.VMEM((N_TOKENS, NUM_EXPERTS), jnp.float32),
            pltpu.VMEM((N_TOKENS, D_IN), jnp.bfloat16),
        ],
    )(x, W, b, gate_W, gate_b, gamma, beta)
